# baseline (speedup 1.0000x reference)
import functools

import jax
import jax.numpy as jnp
from jax import lax
from jax.experimental import pallas as pl
from jax.experimental.pallas import tpu as pltpu
from jax.experimental.pallas import tpu_sc as plsc

B, S, D = 2, 2048, 1024
E, DFF, CAP = 8, 4096, 320
T = B * S
NSLOT = E * B * CAP
TRASH = T
SB = 512
DFB = 1024

NC, NS = 2, 16
SLOTS_PER_TILE = NSLOT // (NC * NS)
NTOK_TILE = T // NS
GRP = 128
DUMP = NSLOT
TBL = NSLOT + 512
CH = 32
NCH = SLOTS_PER_TILE // CH
NBUF = 3


def _router_body(hs_ref, wg_ref, logits_ref, mp_ref, ei_ref, pri_ref, carry_ref):
    j = pl.program_id(1)
    x = hs_ref[0]
    lg = jnp.dot(x, wg_ref[...], preferred_element_type=jnp.float32)
    logits_ref[0] = lg
    m = jnp.max(lg, axis=-1, keepdims=True)
    p = jnp.exp(lg - m)
    s = jnp.sum(p, axis=-1, keepdims=True)
    probs = p / s
    mp = jnp.max(probs, axis=-1, keepdims=True)
    lanes = lax.broadcasted_iota(jnp.int32, (SB, E), 1)
    cand = jnp.where(probs == mp, lanes, E)
    ei = jnp.min(cand, axis=-1, keepdims=True)
    oh = (lanes == ei).astype(jnp.float32)

    @pl.when(j == 0)
    def _():
        carry_ref[...] = jnp.zeros_like(carry_ref)

    r = lax.broadcasted_iota(jnp.int32, (SB, SB), 0)
    c = lax.broadcasted_iota(jnp.int32, (SB, SB), 1)
    tri = (r >= c).astype(jnp.float32)
    cum = jnp.dot(tri, oh, preferred_element_type=jnp.float32)
    cum = cum + carry_ref[0:1, 0:E]
    pri = jnp.sum(oh * cum, axis=-1, keepdims=True)
    carry_ref[0:1, 0:E] = carry_ref[0:1, 0:E] + jnp.sum(oh, axis=0, keepdims=True)

    mp_ref[...] = mp
    ei_ref[...] = ei
    pri_ref[...] = pri.astype(jnp.int32)


_router_call = pl.pallas_call(
    _router_body,
    grid=(B, S // SB),
    in_specs=[
        pl.BlockSpec((1, SB, D), lambda b, j: (b, j, 0)),
        pl.BlockSpec((D, E), lambda b, j: (0, 0)),
    ],
    out_specs=[
        pl.BlockSpec((1, SB, E), lambda b, j: (b, j, 0)),
        pl.BlockSpec((SB, 1), lambda b, j: (b * (S // SB) + j, 0)),
        pl.BlockSpec((SB, 1), lambda b, j: (b * (S // SB) + j, 0)),
        pl.BlockSpec((SB, 1), lambda b, j: (b * (S // SB) + j, 0)),
    ],
    out_shape=[
        jax.ShapeDtypeStruct((B, S, E), jnp.float32),
        jax.ShapeDtypeStruct((T, 1), jnp.float32),
        jax.ShapeDtypeStruct((T, 1), jnp.int32),
        jax.ShapeDtypeStruct((T, 1), jnp.int32),
    ],
    scratch_shapes=[pltpu.VMEM((8, 128), jnp.float32)],
)


WIN = NSLOT // NS


def _dispatch_body(ei_hbm, pri_hbm, mp_hbm, hs_hbm,
                   src_hbm, prob_hbm, xe_hbm,
                   e_v, p_v, q_v,
                   sp1_v, prob_v, so_v, t160_v,
                   i0, i1, i2, i3, i4,
                   r0, r1, r2,
                   sp1_sh,
                   g0, g1, g2, w0, w1, w2, o0, o1):
    cid = lax.axis_index("c")
    sid = lax.axis_index("s")
    idx_bufs = [i0, i1, i2, i3, i4]
    rbufs = [r0, r1, r2]
    gsems = [g0, g1, g2]
    wsems = [w0, w1, w2]
    wbase = sid * WIN

    ld = [
        pltpu.async_copy(ei_hbm, e_v, g0),
        pltpu.async_copy(pri_hbm, p_v, g1),
        pltpu.async_copy(mp_hbm, q_v, g2),
    ]
    for i in range(WIN // 16):
        sp1_v[pl.ds(i * 16, 16)] = jnp.zeros((16,), jnp.int32)
    for d in ld:
        d.wait()

    def tok_body(i, carry):
        sl = pl.ds(i * 16, 16)
        sv = i * 16 + lax.iota(jnp.int32, 16)
        e = e_v[sl]
        p = p_v[sl]
        routed = p <= CAP
        b = jnp.right_shift(sv, 11)
        cslot = jnp.minimum(p - 1, CAP - 1)
        f = e * (B * CAP) + b * CAP + cslot
        mine = routed & (f >= wbase) & (f < wbase + WIN)
        fl = jnp.where(mine, f - wbase, 0)
        plsc.store_scatter(sp1_v, [fl], sv + 1, mask=mine)
        plsc.store_scatter(prob_v, [fl], q_v[sl], mask=mine)
        return carry

    lax.fori_loop(0, T // 16, tok_body, 0)

    pub = pltpu.async_copy(sp1_v, sp1_sh.at[pl.ds(wbase, WIN)], o0)

    for i in range(WIN // 16):
        sl = pl.ds(i * 16, 16)
        v = sp1_v[sl]
        slot = wbase + i * 16 + lax.iota(jnp.int32, 16)
        trash = T + lax.div(slot, SLOTS_PER_TILE)
        so_v[sl] = jnp.where(v == 0, trash, v - 1)

    @pl.when(cid == 0)
    def _():
        d3 = pltpu.async_copy(so_v, src_hbm.at[pl.ds(wbase, WIN)], w0)
        d4 = pltpu.async_copy(prob_v, prob_hbm.at[pl.ds(wbase, WIN)], w1)
        d3.wait()
        d4.wait()

    pub.wait()
    plsc.subcore_barrier()

    gbase = cid * (NSLOT // NC) + sid * SLOTS_PER_TILE
    pltpu.sync_copy(sp1_sh.at[pl.ds(gbase, SLOTS_PER_TILE)], t160_v)
    for k in range(NCH):
        for i in range(CH // 16):
            sl = pl.ds(i * 16, 16)
            idx_bufs[k][sl] = jnp.maximum(t160_v[pl.ds(k * CH + i * 16, 16)] - 1, 0)

    gd = [None] * NCH
    wd = [None] * NCH
    for k in range(NBUF):
        gd[k] = pltpu.async_copy(hs_hbm.at[idx_bufs[k]], rbufs[k], gsems[k])
    for k in range(NCH):
        gd[k].wait()
        wd[k] = pltpu.async_copy(
            rbufs[k % NBUF], xe_hbm.at[pl.ds(gbase + k * CH, CH)], wsems[k % NBUF])
        nk = k + NBUF
        if nk < NCH:
            wd[k].wait()
            gd[nk] = pltpu.async_copy(hs_hbm.at[idx_bufs[nk]], rbufs[nk % NBUF],
                                      gsems[nk % NBUF])
    for k in range(NCH - NBUF, NCH):
        wd[k].wait()


@functools.lru_cache(maxsize=None)
def _get_dispatch_call():
    return functools.partial(
        pl.kernel,
        out_type=[
            jax.ShapeDtypeStruct((NSLOT,), jnp.int32),
            jax.ShapeDtypeStruct((NSLOT,), jnp.float32),
            jax.ShapeDtypeStruct((NSLOT, D), jnp.float32),
        ],
        mesh=plsc.VectorSubcoreMesh(
            core_axis_name="c", subcore_axis_name="s",
            num_cores=NC, num_subcores=NS),
        scratch_types=[
            pltpu.VMEM((T,), jnp.int32),
            pltpu.VMEM((T,), jnp.int32),
            pltpu.VMEM((T,), jnp.float32),
            pltpu.VMEM((WIN,), jnp.int32),
            pltpu.VMEM((WIN,), jnp.float32),
            pltpu.VMEM((WIN,), jnp.int32),
            pltpu.VMEM((SLOTS_PER_TILE,), jnp.int32),
            pltpu.VMEM((CH,), jnp.int32),
            pltpu.VMEM((CH,), jnp.int32),
            pltpu.VMEM((CH,), jnp.int32),
            pltpu.VMEM((CH,), jnp.int32),
            pltpu.VMEM((CH,), jnp.int32),
            pltpu.VMEM((CH, D), jnp.float32),
            pltpu.VMEM((CH, D), jnp.float32),
            pltpu.VMEM((CH, D), jnp.float32),
            pltpu.VMEM_SHARED((NSLOT,), jnp.int32),
            pltpu.SemaphoreType.DMA,
            pltpu.SemaphoreType.DMA,
            pltpu.SemaphoreType.DMA,
            pltpu.SemaphoreType.DMA,
            pltpu.SemaphoreType.DMA,
            pltpu.SemaphoreType.DMA,
            pltpu.SemaphoreType.DMA,
            pltpu.SemaphoreType.DMA,
        ],
        compiler_params=pltpu.CompilerParams(needs_layout_passes=False),
    )(_dispatch_body)


def _ffn_body(x_ref, w1_ref, w2_ref, p_ref, y_ref):
    k = pl.program_id(1)
    nk = pl.num_programs(1)
    x16 = x_ref[...].astype(jnp.bfloat16)
    w116 = w1_ref[0].astype(jnp.bfloat16)
    w216 = w2_ref[0].astype(jnp.bfloat16)
    h = jnp.dot(x16, w116, preferred_element_type=jnp.float32)
    h = h * (1.0 / (1.0 + jnp.exp(-h)))
    part = jnp.dot(h.astype(jnp.bfloat16), w216,
                   preferred_element_type=jnp.float32)

    @pl.when(k == 0)
    def _():
        y_ref[...] = jnp.zeros_like(y_ref)

    y_ref[...] += part

    @pl.when(k == nk - 1)
    def _():
        y_ref[...] = y_ref[...] * p_ref[...]


_ffn_call = pl.pallas_call(
    _ffn_body,
    grid=(E, DFF // DFB),
    in_specs=[
        pl.BlockSpec((B * CAP, D), lambda e, k: (e, 0)),
        pl.BlockSpec((1, D, DFB), lambda e, k: (e, 0, k)),
        pl.BlockSpec((1, DFB, D), lambda e, k: (e, k, 0)),
        pl.BlockSpec((B * CAP, 1), lambda e, k: (e, 0)),
    ],
    out_specs=pl.BlockSpec((B * CAP, D), lambda e, k: (e, 0)),
    out_shape=jax.ShapeDtypeStruct((NSLOT, D), jnp.float32),
)


def _combine_body(y_hbm, src_hbm, out_hbm,
                  i0, i1, i2, i3, i4,
                  r0, r1, r2,
                  g0, g1, g2, w0, w1, w2):
    cid = lax.axis_index("c")
    sid = lax.axis_index("s")
    idx_bufs = [i0, i1, i2, i3, i4]
    rbufs = [r0, r1, r2]
    gsems = [g0, g1, g2]
    wsems = [w0, w1, w2]
    base = cid * (NSLOT // NC) + sid * SLOTS_PER_TILE

    for k in range(NCH):
        pltpu.sync_copy(src_hbm.at[pl.ds(base + k * CH, CH)], idx_bufs[k])

    gd = [None] * NCH
    wd = [None] * NCH
    for k in range(NBUF):
        gd[k] = pltpu.async_copy(y_hbm.at[pl.ds(base + k * CH, CH)], rbufs[k], gsems[k])
    for k in range(NCH):
        gd[k].wait()
        wd[k] = pltpu.async_copy(rbufs[k % NBUF], out_hbm.at[idx_bufs[k]], wsems[k % NBUF])
        nk = k + NBUF
        if nk < NCH:
            wd[k].wait()
            gd[nk] = pltpu.async_copy(y_hbm.at[pl.ds(base + nk * CH, CH)],
                                      rbufs[nk % NBUF], gsems[nk % NBUF])
    for k in range(NCH - NBUF, NCH):
        wd[k].wait()


@functools.lru_cache(maxsize=None)
def _get_combine_call():
    return functools.partial(
        pl.kernel,
        out_type=jax.ShapeDtypeStruct((T + 2 * NC * NS, D), jnp.float32),
        mesh=plsc.VectorSubcoreMesh(
            core_axis_name="c", subcore_axis_name="s",
            num_cores=NC, num_subcores=NS),
        scratch_types=[
            pltpu.VMEM((CH,), jnp.int32),
            pltpu.VMEM((CH,), jnp.int32),
            pltpu.VMEM((CH,), jnp.int32),
            pltpu.VMEM((CH,), jnp.int32),
            pltpu.VMEM((CH,), jnp.int32),
            pltpu.VMEM((CH, D), jnp.float32),
            pltpu.VMEM((CH, D), jnp.float32),
            pltpu.VMEM((CH, D), jnp.float32),
            pltpu.SemaphoreType.DMA,
            pltpu.SemaphoreType.DMA,
            pltpu.SemaphoreType.DMA,
            pltpu.SemaphoreType.DMA,
            pltpu.SemaphoreType.DMA,
            pltpu.SemaphoreType.DMA,
        ],
        compiler_params=pltpu.CompilerParams(needs_layout_passes=False),
    )(_combine_body)


def _merge_body(hs_ref, yr_ref, mp_ref, pri_ref, o_ref):
    sel = pri_ref[...] <= CAP
    o_ref[...] = jnp.where(sel, yr_ref[...], mp_ref[...] * hs_ref[...])


_merge_call = pl.pallas_call(
    _merge_body,
    grid=(T // 256,),
    in_specs=[
        pl.BlockSpec((256, D), lambda i: (i, 0)),
        pl.BlockSpec((256, D), lambda i: (i, 0)),
        pl.BlockSpec((256, 1), lambda i: (i, 0)),
        pl.BlockSpec((256, 1), lambda i: (i, 0)),
    ],
    out_specs=pl.BlockSpec((256, D), lambda i: (i, 0)),
    out_shape=jax.ShapeDtypeStruct((T, D), jnp.float32),
)


def kernel(hidden_states, w_gate, w1, w2):
    hs_flat = hidden_states.reshape(T, D)
    wg_t = w_gate.T
    logits, mp, ei, pri = _router_call(hidden_states, wg_t)
    src, prob, x_e = _get_dispatch_call()(
        ei.reshape(T), pri.reshape(T), mp.reshape(T), hs_flat)
    y = _ffn_call(x_e, w1, w2, prob.reshape(NSLOT, 1))
    out_routed = _get_combine_call()(y, src)
    out = _merge_call(hs_flat, out_routed, mp, pri)
    return out.reshape(B, S, D), logits


# speedup vs baseline: 1.0429x; 1.0429x over previous
import functools

import jax
import jax.numpy as jnp
from jax import lax
from jax.experimental import pallas as pl
from jax.experimental.pallas import tpu as pltpu
from jax.experimental.pallas import tpu_sc as plsc

B, S, D = 2, 2048, 1024
E, DFF, CAP = 8, 4096, 320
T = B * S
NSLOT = E * B * CAP
SB = 1024
DFB = 2048

NC, NS = 2, 16
SLOTS_PER_TILE = NSLOT // (NC * NS)
NTOK_TILE = T // NS
GRP = 128
DUMP = NSLOT
TBL = NSLOT + 512
CH = 32
NCH = SLOTS_PER_TILE // CH
NBUF = 3


def _router_body(hs_ref, wg_ref, logits_ref, mp_ref, ei_ref, pri_ref, carry_ref):
    j = pl.program_id(1)
    x = hs_ref[0]
    lg = jnp.dot(x, wg_ref[...], preferred_element_type=jnp.float32)
    logits_ref[0] = lg
    m = jnp.max(lg, axis=-1, keepdims=True)
    p = jnp.exp(lg - m)
    s = jnp.sum(p, axis=-1, keepdims=True)
    probs = p / s
    mp = jnp.max(probs, axis=-1, keepdims=True)
    lanes = lax.broadcasted_iota(jnp.int32, (SB, E), 1)
    cand = jnp.where(probs == mp, lanes, E)
    ei = jnp.min(cand, axis=-1, keepdims=True)
    oh = (lanes == ei).astype(jnp.float32)

    @pl.when(j == 0)
    def _():
        carry_ref[...] = jnp.zeros_like(carry_ref)

    r = lax.broadcasted_iota(jnp.int32, (SB, SB), 0)
    c = lax.broadcasted_iota(jnp.int32, (SB, SB), 1)
    tri = (r >= c).astype(jnp.float32)
    cum = jnp.dot(tri, oh, preferred_element_type=jnp.float32)
    cum = cum + carry_ref[0:1, 0:E]
    pri = jnp.sum(oh * cum, axis=-1, keepdims=True)
    carry_ref[0:1, 0:E] = carry_ref[0:1, 0:E] + jnp.sum(oh, axis=0, keepdims=True)

    mp_ref[...] = mp
    ei_ref[...] = ei
    pri_ref[...] = pri.astype(jnp.int32)


_router_call = pl.pallas_call(
    _router_body,
    grid=(B, S // SB),
    in_specs=[
        pl.BlockSpec((1, SB, D), lambda b, j: (b, j, 0)),
        pl.BlockSpec((D, E), lambda b, j: (0, 0)),
    ],
    out_specs=[
        pl.BlockSpec((1, SB, E), lambda b, j: (b, j, 0)),
        pl.BlockSpec((SB, 1), lambda b, j: (b * (S // SB) + j, 0)),
        pl.BlockSpec((SB, 1), lambda b, j: (b * (S // SB) + j, 0)),
        pl.BlockSpec((SB, 1), lambda b, j: (b * (S // SB) + j, 0)),
    ],
    out_shape=[
        jax.ShapeDtypeStruct((B, S, E), jnp.float32),
        jax.ShapeDtypeStruct((T, 1), jnp.float32),
        jax.ShapeDtypeStruct((T, 1), jnp.int32),
        jax.ShapeDtypeStruct((T, 1), jnp.int32),
    ],
    scratch_shapes=[pltpu.VMEM((8, 128), jnp.float32)],
)


WIN = NSLOT // NS


def _dispatch_body(ei_hbm, pri_hbm, mp_hbm, hs_hbm,
                   src_hbm, prob_hbm, xe_hbm,
                   e_v, p_v, q_v,
                   sp1_v, prob_v, so_v, t160_v,
                   i0, i1, i2, i3, i4,
                   r0, r1, r2,
                   sp1_sh,
                   g0, g1, g2, w0, w1, w2, o0, o1):
    cid = lax.axis_index("c")
    sid = lax.axis_index("s")
    idx_bufs = [i0, i1, i2, i3, i4]
    rbufs = [r0, r1, r2]
    gsems = [g0, g1, g2]
    wsems = [w0, w1, w2]
    wbase = sid * WIN

    ld = [
        pltpu.async_copy(ei_hbm, e_v, g0),
        pltpu.async_copy(pri_hbm, p_v, g1),
        pltpu.async_copy(mp_hbm, q_v, g2),
    ]
    for i in range(WIN // 16):
        sp1_v[pl.ds(i * 16, 16)] = jnp.zeros((16,), jnp.int32)
    for d in ld:
        d.wait()

    def tok_body(i, carry):
        sl = pl.ds(i * 16, 16)
        sv = i * 16 + lax.iota(jnp.int32, 16)
        e = e_v[sl]
        p = p_v[sl]
        routed = p <= CAP
        b = jnp.right_shift(sv, 11)
        cslot = jnp.minimum(p - 1, CAP - 1)
        f = e * (B * CAP) + b * CAP + cslot
        mine = routed & (f >= wbase) & (f < wbase + WIN)
        fl = jnp.where(mine, f - wbase, 0)
        plsc.store_scatter(sp1_v, [fl], sv + 1, mask=mine)
        plsc.store_scatter(prob_v, [fl], q_v[sl], mask=mine)
        return carry

    lax.fori_loop(0, T // 16, tok_body, 0)

    pub = pltpu.async_copy(sp1_v, sp1_sh.at[pl.ds(wbase, WIN)], o0)

    for i in range(WIN // 16):
        sl = pl.ds(i * 16, 16)
        v = sp1_v[sl]
        slot = wbase + i * 16 + lax.iota(jnp.int32, 16)
        trash = T + lax.div(slot, SLOTS_PER_TILE)
        so_v[sl] = jnp.where(v == 0, trash, v - 1)

    @pl.when(cid == 0)
    def _():
        d3 = pltpu.async_copy(so_v, src_hbm.at[pl.ds(wbase, WIN)], w0)
        d4 = pltpu.async_copy(prob_v, prob_hbm.at[pl.ds(wbase, WIN)], w1)
        d3.wait()
        d4.wait()

    pub.wait()
    plsc.subcore_barrier()

    gbase = cid * (NSLOT // NC) + sid * SLOTS_PER_TILE
    pltpu.sync_copy(sp1_sh.at[pl.ds(gbase, SLOTS_PER_TILE)], t160_v)
    for k in range(NCH):
        for i in range(CH // 16):
            sl = pl.ds(i * 16, 16)
            idx_bufs[k][sl] = jnp.maximum(t160_v[pl.ds(k * CH + i * 16, 16)] - 1, 0)

    gd = [None] * NCH
    wd = [None] * NCH
    for k in range(NBUF):
        gd[k] = pltpu.async_copy(hs_hbm.at[idx_bufs[k]], rbufs[k], gsems[k])
    for k in range(NCH):
        gd[k].wait()
        wd[k] = pltpu.async_copy(
            rbufs[k % NBUF], xe_hbm.at[pl.ds(gbase + k * CH, CH)], wsems[k % NBUF])
        nk = k + NBUF
        if nk < NCH:
            wd[k].wait()
            gd[nk] = pltpu.async_copy(hs_hbm.at[idx_bufs[nk]], rbufs[nk % NBUF],
                                      gsems[nk % NBUF])
    for k in range(NCH - NBUF, NCH):
        wd[k].wait()


@functools.lru_cache(maxsize=None)
def _get_dispatch_call():
    return functools.partial(
        pl.kernel,
        out_type=[
            jax.ShapeDtypeStruct((NSLOT,), jnp.int32),
            jax.ShapeDtypeStruct((NSLOT,), jnp.float32),
            jax.ShapeDtypeStruct((NSLOT, D), jnp.float32),
        ],
        mesh=plsc.VectorSubcoreMesh(
            core_axis_name="c", subcore_axis_name="s",
            num_cores=NC, num_subcores=NS),
        scratch_types=[
            pltpu.VMEM((T,), jnp.int32),
            pltpu.VMEM((T,), jnp.int32),
            pltpu.VMEM((T,), jnp.float32),
            pltpu.VMEM((WIN,), jnp.int32),
            pltpu.VMEM((WIN,), jnp.float32),
            pltpu.VMEM((WIN,), jnp.int32),
            pltpu.VMEM((SLOTS_PER_TILE,), jnp.int32),
            pltpu.VMEM((CH,), jnp.int32),
            pltpu.VMEM((CH,), jnp.int32),
            pltpu.VMEM((CH,), jnp.int32),
            pltpu.VMEM((CH,), jnp.int32),
            pltpu.VMEM((CH,), jnp.int32),
            pltpu.VMEM((CH, D), jnp.float32),
            pltpu.VMEM((CH, D), jnp.float32),
            pltpu.VMEM((CH, D), jnp.float32),
            pltpu.VMEM_SHARED((NSLOT,), jnp.int32),
            pltpu.SemaphoreType.DMA,
            pltpu.SemaphoreType.DMA,
            pltpu.SemaphoreType.DMA,
            pltpu.SemaphoreType.DMA,
            pltpu.SemaphoreType.DMA,
            pltpu.SemaphoreType.DMA,
            pltpu.SemaphoreType.DMA,
            pltpu.SemaphoreType.DMA,
        ],
        compiler_params=pltpu.CompilerParams(needs_layout_passes=False),
    )(_dispatch_body)


def _ffn_body(x_ref, w1_ref, w2_ref, p_ref, y_ref):
    k = pl.program_id(1)
    nk = pl.num_programs(1)
    x16 = x_ref[...].astype(jnp.bfloat16)
    w116 = w1_ref[0].astype(jnp.bfloat16)
    w216 = w2_ref[0].astype(jnp.bfloat16)
    h = jnp.dot(x16, w116, preferred_element_type=jnp.float32)
    h = h * (1.0 / (1.0 + jnp.exp(-h)))
    part = jnp.dot(h.astype(jnp.bfloat16), w216,
                   preferred_element_type=jnp.float32)

    @pl.when(k == 0)
    def _():
        y_ref[...] = jnp.zeros_like(y_ref)

    y_ref[...] += part

    @pl.when(k == nk - 1)
    def _():
        y_ref[...] = y_ref[...] * p_ref[...]


_ffn_call = pl.pallas_call(
    _ffn_body,
    grid=(E, DFF // DFB),
    in_specs=[
        pl.BlockSpec((B * CAP, D), lambda e, k: (e, 0)),
        pl.BlockSpec((1, D, DFB), lambda e, k: (e, 0, k)),
        pl.BlockSpec((1, DFB, D), lambda e, k: (e, k, 0)),
        pl.BlockSpec((B * CAP, 1), lambda e, k: (e, 0)),
    ],
    out_specs=pl.BlockSpec((B * CAP, D), lambda e, k: (e, 0)),
    out_shape=jax.ShapeDtypeStruct((NSLOT, D), jnp.float32),
)


def _combine_body(y_hbm, src_hbm, out_hbm,
                  i0, i1, i2, i3, i4,
                  r0, r1, r2,
                  g0, g1, g2, w0, w1, w2):
    cid = lax.axis_index("c")
    sid = lax.axis_index("s")
    idx_bufs = [i0, i1, i2, i3, i4]
    rbufs = [r0, r1, r2]
    gsems = [g0, g1, g2]
    wsems = [w0, w1, w2]
    base = cid * (NSLOT // NC) + sid * SLOTS_PER_TILE

    for k in range(NCH):
        pltpu.sync_copy(src_hbm.at[pl.ds(base + k * CH, CH)], idx_bufs[k])

    gd = [None] * NCH
    wd = [None] * NCH
    for k in range(NBUF):
        gd[k] = pltpu.async_copy(y_hbm.at[pl.ds(base + k * CH, CH)], rbufs[k], gsems[k])
    for k in range(NCH):
        gd[k].wait()
        wd[k] = pltpu.async_copy(rbufs[k % NBUF], out_hbm.at[idx_bufs[k]], wsems[k % NBUF])
        nk = k + NBUF
        if nk < NCH:
            wd[k].wait()
            gd[nk] = pltpu.async_copy(y_hbm.at[pl.ds(base + nk * CH, CH)],
                                      rbufs[nk % NBUF], gsems[nk % NBUF])
    for k in range(NCH - NBUF, NCH):
        wd[k].wait()


@functools.lru_cache(maxsize=None)
def _get_combine_call():
    return functools.partial(
        pl.kernel,
        out_type=jax.ShapeDtypeStruct((T + 2 * NC * NS, D), jnp.float32),
        mesh=plsc.VectorSubcoreMesh(
            core_axis_name="c", subcore_axis_name="s",
            num_cores=NC, num_subcores=NS),
        scratch_types=[
            pltpu.VMEM((CH,), jnp.int32),
            pltpu.VMEM((CH,), jnp.int32),
            pltpu.VMEM((CH,), jnp.int32),
            pltpu.VMEM((CH,), jnp.int32),
            pltpu.VMEM((CH,), jnp.int32),
            pltpu.VMEM((CH, D), jnp.float32),
            pltpu.VMEM((CH, D), jnp.float32),
            pltpu.VMEM((CH, D), jnp.float32),
            pltpu.SemaphoreType.DMA,
            pltpu.SemaphoreType.DMA,
            pltpu.SemaphoreType.DMA,
            pltpu.SemaphoreType.DMA,
            pltpu.SemaphoreType.DMA,
            pltpu.SemaphoreType.DMA,
        ],
        compiler_params=pltpu.CompilerParams(needs_layout_passes=False),
    )(_combine_body)


def _merge_body(hs_ref, yr_ref, mp_ref, pri_ref, o_ref):
    sel = pri_ref[...] <= CAP
    o_ref[...] = jnp.where(sel, yr_ref[...], mp_ref[...] * hs_ref[...])


_merge_call = pl.pallas_call(
    _merge_body,
    grid=(T // 256,),
    in_specs=[
        pl.BlockSpec((256, D), lambda i: (i, 0)),
        pl.BlockSpec((256, D), lambda i: (i, 0)),
        pl.BlockSpec((256, 1), lambda i: (i, 0)),
        pl.BlockSpec((256, 1), lambda i: (i, 0)),
    ],
    out_specs=pl.BlockSpec((256, D), lambda i: (i, 0)),
    out_shape=jax.ShapeDtypeStruct((T, D), jnp.float32),
)


def kernel(hidden_states, w_gate, w1, w2):
    hs_flat = hidden_states.reshape(T, D)
    wg_t = w_gate.T
    logits, mp, ei, pri = _router_call(hidden_states, wg_t)
    src, prob, x_e = _get_dispatch_call()(
        ei.reshape(T), pri.reshape(T), mp.reshape(T), hs_flat)
    y = _ffn_call(x_e, w1, w2, prob.reshape(NSLOT, 1))
    out_routed = _get_combine_call()(y, src)
    out = _merge_call(hs_flat, out_routed, mp, pri)
    return out.reshape(B, S, D), logits
